# baseline (speedup 1.0000x reference)
import jax
import jax.numpy as jnp
import numpy as np
from jax.experimental import pallas as pl

_CODE_VALUES = np.array([-3.0, -1.0, 1.0, 3.0], dtype=np.float32)


def _codebook_np():
    rs = np.random.RandomState(0)
    idx = rs.randint(0, 4, size=(512, 8))
    return _CODE_VALUES[idx]


def _build_consts():
    import ml_dtypes
    cb = _codebook_np()
    cnorm = (cb * cb).sum(axis=1)
    w1t = np.zeros((2048, 33), dtype=np.float32)
    w2t = np.zeros((32, 2048), dtype=np.float32)
    for g in range(4):
        w1t[512 * g : 512 * (g + 1), 8 * g : 8 * g + 8] = -2.0 * cb
        w1t[512 * g : 512 * (g + 1), 32] = cnorm
        w2t[8 * g : 8 * g + 8, 512 * g : 512 * (g + 1)] = cb.T
    return (w1t.astype(ml_dtypes.bfloat16), w2t.astype(ml_dtypes.bfloat16))


_W1T, _W2T = _build_consts()

_TB = 4096
_NB = (4096 * 4096) // 32
_BIG = np.float32(1e9)


def _quant_body(x_ref, w1t_ref, w2t_ref, o_ref):
    x = x_ref[...]
    xT3 = x.T.reshape(4, 32, x.shape[0])
    xt = jnp.concatenate([xT3[0], xT3[1], xT3[2], xT3[3]], axis=1)
    d = jnp.maximum(jnp.max(jnp.abs(xt), axis=0, keepdims=True) / 3.0, 1e-8)
    sub = (xt / d).astype(jnp.bfloat16)
    sub1 = jnp.concatenate(
        [sub, jnp.ones((1, sub.shape[1]), jnp.bfloat16)], axis=0)
    dist = jax.lax.dot_general(
        w1t_ref[...], sub1, (((1,), (0,)), ((), ())),
        preferred_element_type=jnp.float32,
    )
    tb = dist.shape[1]
    iota = jax.lax.broadcasted_iota(
        jnp.int32, (512, tb), 0).astype(jnp.float32)
    ohs = []
    for g in range(4):
        sg = dist[512 * g : 512 * (g + 1), :]
        m = jnp.min(sg, axis=0, keepdims=True)
        acc = jnp.full((8, tb), _BIG, jnp.float32)
        for k in range(64):
            ck = jnp.where(sg[8 * k : 8 * k + 8, :] == m,
                           iota[8 * k : 8 * k + 8, :], _BIG)
            acc = jnp.minimum(acc, ck)
        idx = jnp.min(acc, axis=0, keepdims=True)
        ohs.append((iota == idx).astype(jnp.bfloat16))
    oh = jnp.concatenate(ohs, axis=0)
    q = jax.lax.dot_general(
        w2t_ref[...], oh, (((1,), (0,)), ((), ())),
        preferred_element_type=jnp.float32,
    )
    deq = q * d
    out = xt + (deq - xt)
    tr = x.shape[0]
    o_ref[...] = jnp.concatenate(
        [out[:, g * tr : (g + 1) * tr] for g in range(4)], axis=0).T


@jax.jit
def kernel(w):
    nr = _NB // 4
    tr = _TB // 4
    wf = w.reshape(nr, 128)
    grid = nr // tr
    out = pl.pallas_call(
        _quant_body,
        grid=(grid,),
        in_specs=[
            pl.BlockSpec((tr, 128), lambda i: (i, 0)),
            pl.BlockSpec((2048, 33), lambda i: (0, 0)),
            pl.BlockSpec((32, 2048), lambda i: (0, 0)),
        ],
        out_specs=pl.BlockSpec((tr, 128), lambda i: (i, 0)),
        out_shape=jax.ShapeDtypeStruct((nr, 128), jnp.float32),
    )(wf, jnp.asarray(_W1T), jnp.asarray(_W2T))
    return out.reshape(w.shape)
